# Initial kernel scaffold; baseline (speedup 1.0000x reference)
import jax
import jax.numpy as jnp
from jax.experimental import pallas as pl


def kernel(relation_logits_raw, rel_labels, centers):
    raise NotImplementedError("write your pallas kernel here")


# speedup vs baseline: 1.0966x; 1.0966x over previous
import functools

import jax
import jax.numpy as jnp
from jax import lax
from jax.experimental import pallas as pl
from jax.experimental.pallas import tpu as pltpu
from jax.experimental.pallas import tpu_sc as plsc

N_CLASSES = 51
D = 576
B = 4096
LAMBDA = 0.03
EPS = 0.09

NC, NS, L = 2, 16, 16
NW = NC * NS
ROWS_PER_W = B // NW
CHUNK = 64
N_CHUNKS = ROWS_PER_W // CHUNK
D_VECS = D // L


def _sc_body(x_hbm, lab_hbm, cen_hbm, out_hbm, idx_v, xbuf, cbuf, accv, sem):
    wid = lax.axis_index("s") * NC + lax.axis_index("c")
    base = wid * ROWS_PER_W

    acc = (jnp.zeros((L,), jnp.float32),) * 4
    for t in range(N_CHUNKS):
        row0 = base + t * CHUNK
        pltpu.sync_copy(lab_hbm.at[pl.ds(row0, CHUNK)], idx_v)
        cp_x = pltpu.async_copy(x_hbm.at[pl.ds(row0, CHUNK)], xbuf, sem)
        cp_c = pltpu.async_copy(cen_hbm.at[idx_v], cbuf, sem)
        cp_x.wait()
        cp_c.wait()

        def row_body(r, a):
            a0, a1, a2, a3 = a
            accs = [a0, a1, a2, a3]
            for j in range(D_VECS):
                dv = xbuf[r, pl.ds(j * L, L)] - cbuf[r, pl.ds(j * L, L)]
                accs[j % 4] = accs[j % 4] + dv * dv
            return tuple(accs)

        acc = lax.fori_loop(0, CHUNK, row_body, acc)

    accv[...] = (acc[0] + acc[1]) + (acc[2] + acc[3])
    pltpu.sync_copy(accv, out_hbm.at[wid])


@functools.cache
def _sc_loss():
    return pl.kernel(
        _sc_body,
        out_type=jax.ShapeDtypeStruct((NW, L), jnp.float32),
        mesh=plsc.VectorSubcoreMesh(core_axis_name="c", subcore_axis_name="s"),
        scratch_types=[
            pltpu.VMEM((CHUNK,), jnp.int32),
            pltpu.VMEM((CHUNK, D), jnp.float32),
            pltpu.VMEM((CHUNK, D), jnp.float32),
            pltpu.VMEM((L,), jnp.float32),
            pltpu.SemaphoreType.DMA,
        ],
        compiler_params=pltpu.CompilerParams(use_tc_tiling_on_sc=False),
    )


def _tc_body(c_ref, p_ref, w_ref, l_ref):
    c = c_ref[...]
    sq = jnp.sum(c * c, axis=1)
    g = lax.dot_general(c, c, (((1,), (1,)), ((), ())),
                        preferred_element_type=jnp.float32)
    d2 = sq[:, None] + sq[None, :] - 2.0 * g
    dist = jnp.sqrt(jnp.maximum(d2, 0.0))
    gc = jnp.sum(dist, axis=1) * (1.0 / N_CLASSES)
    mx = jnp.max(gc)
    mn = jnp.min(gc)
    w_ref[...] = (gc - mn + EPS) / (mx - mn)
    loss = jnp.sum(p_ref[...]) * (LAMBDA / (B * D))
    l_ref[...] = jnp.reshape(loss, (1, 1))


_tc_pair = pl.pallas_call(
    _tc_body,
    out_shape=(
        jax.ShapeDtypeStruct((N_CLASSES,), jnp.float32),
        jax.ShapeDtypeStruct((1, 1), jnp.float32),
    ),
)


def kernel(relation_logits_raw, rel_labels, centers):
    labels = rel_labels.astype(jnp.int32)
    parts = _sc_loss()(relation_logits_raw, labels, centers)
    weight, loss = _tc_pair(centers, parts)
    return (weight, loss[0, 0], relation_logits_raw)


# speedup vs baseline: 1.1531x; 1.0515x over previous
import functools

import jax
import jax.numpy as jnp
from jax import lax
from jax.experimental import pallas as pl
from jax.experimental.pallas import tpu as pltpu
from jax.experimental.pallas import tpu_sc as plsc

N_CLASSES = 51
D = 576
B = 4096
LAMBDA = 0.03
EPS = 0.09

NC, NS, L = 2, 16, 16
NW = NC * NS
ROWS_PER_W = B // NW
CHUNK = 32
N_CHUNKS = ROWS_PER_W // CHUNK
D_VECS = D // L


def _sc_body(x_hbm, lab_hbm, cen_hbm, out_hbm,
             idx_v, xb0, xb1, cb0, cb1, accv, sx0, sx1, sc0, sc1):
    wid = lax.axis_index("s") * NC + lax.axis_index("c")
    base = wid * ROWS_PER_W
    pltpu.sync_copy(lab_hbm.at[pl.ds(base, ROWS_PER_W)], idx_v)

    xbufs, cbufs = (xb0, xb1), (cb0, cb1)
    sxs, scs = (sx0, sx1), (sc0, sc1)

    def start(t):
        row0 = base + t * CHUNK
        cpx = pltpu.async_copy(x_hbm.at[pl.ds(row0, CHUNK)], xbufs[t % 2],
                               sxs[t % 2])
        cpc = pltpu.async_copy(cen_hbm.at[idx_v.at[pl.ds(t * CHUNK, CHUNK)]],
                               cbufs[t % 2], scs[t % 2])
        return cpx, cpc

    inflight = start(0)
    acc = (jnp.zeros((L,), jnp.float32),) * 4
    for t in range(N_CHUNKS):
        nxt = start(t + 1) if t + 1 < N_CHUNKS else None
        inflight[0].wait()
        inflight[1].wait()
        xb, cb = xbufs[t % 2], cbufs[t % 2]

        def row_body(r, a):
            accs = list(a)
            for j in range(D_VECS):
                dv = xb[r, pl.ds(j * L, L)] - cb[r, pl.ds(j * L, L)]
                accs[j % 4] = accs[j % 4] + dv * dv
            return tuple(accs)

        acc = lax.fori_loop(0, CHUNK, row_body, acc)
        inflight = nxt

    accv[...] = (acc[0] + acc[1]) + (acc[2] + acc[3])
    pltpu.sync_copy(accv, out_hbm.at[wid])


@functools.cache
def _sc_loss():
    return pl.kernel(
        _sc_body,
        out_type=jax.ShapeDtypeStruct((NW, L), jnp.float32),
        mesh=plsc.VectorSubcoreMesh(core_axis_name="c", subcore_axis_name="s"),
        scratch_types=[
            pltpu.VMEM((ROWS_PER_W,), jnp.int32),
            pltpu.VMEM((CHUNK, D), jnp.float32),
            pltpu.VMEM((CHUNK, D), jnp.float32),
            pltpu.VMEM((CHUNK, D), jnp.float32),
            pltpu.VMEM((CHUNK, D), jnp.float32),
            pltpu.VMEM((L,), jnp.float32),
            pltpu.SemaphoreType.DMA,
            pltpu.SemaphoreType.DMA,
            pltpu.SemaphoreType.DMA,
            pltpu.SemaphoreType.DMA,
        ],
        compiler_params=pltpu.CompilerParams(use_tc_tiling_on_sc=False),
    )


def _tc_body(c_ref, p_ref, w_ref, l_ref):
    c = c_ref[...]
    sq = jnp.sum(c * c, axis=1)
    g = lax.dot_general(c, c, (((1,), (1,)), ((), ())),
                        preferred_element_type=jnp.float32,
                        precision=lax.Precision.HIGHEST)
    d2 = sq[:, None] + sq[None, :] - 2.0 * g
    dist = jnp.sqrt(jnp.maximum(d2, 0.0))
    gc = jnp.sum(dist, axis=1) * (1.0 / N_CLASSES)
    mx = jnp.max(gc)
    mn = jnp.min(gc)
    w_ref[...] = (gc - mn + EPS) / (mx - mn)
    loss = jnp.sum(p_ref[...]) * (LAMBDA / (B * D))
    l_ref[...] = jnp.reshape(loss, (1, 1))


_tc_pair = pl.pallas_call(
    _tc_body,
    out_shape=(
        jax.ShapeDtypeStruct((N_CLASSES,), jnp.float32),
        jax.ShapeDtypeStruct((1, 1), jnp.float32),
    ),
)


def _copy_body(x_ref, o_ref):
    o_ref[...] = x_ref[...]


_COPY_ROWS = 512

_tc_copy = pl.pallas_call(
    _copy_body,
    grid=(B // _COPY_ROWS,),
    in_specs=[pl.BlockSpec((_COPY_ROWS, D), lambda i: (i, 0))],
    out_specs=pl.BlockSpec((_COPY_ROWS, D), lambda i: (i, 0)),
    out_shape=jax.ShapeDtypeStruct((B, D), jnp.float32),
)


def kernel(relation_logits_raw, rel_labels, centers):
    labels = rel_labels.astype(jnp.int32)
    parts = _sc_loss()(relation_logits_raw, labels, centers)
    weight, loss = _tc_pair(centers, parts)
    rel_features = _tc_copy(relation_logits_raw)
    return (weight, loss[0, 0], rel_features)


# speedup vs baseline: 1.7311x; 1.5013x over previous
import functools

import jax
import jax.numpy as jnp
from jax import lax
from jax.experimental import pallas as pl
from jax.experimental.pallas import tpu as pltpu
from jax.experimental.pallas import tpu_sc as plsc

N_CLASSES = 51
D = 576
B = 4096
LAMBDA = 0.03
EPS = 0.09

NC, NS, L = 2, 16, 16
NW = NC * NS
C_CHUNKS = 8
B_CHUNKS = NW // C_CHUNKS
CW = D // C_CHUNKS
BW = B // B_CHUNKS
B_VECS = BW // L


def _sc_body(xt_hbm, lab_hbm, cfl_hbm, out_hbm, rel_hbm,
             idx_v, xbuf, ctbuf, accv):
    wid = lax.axis_index("s") * NC + lax.axis_index("c")
    cc = wid % C_CHUNKS
    bc = wid // C_CHUNKS
    c0 = cc * CW
    b0 = bc * BW

    pltpu.sync_copy(lab_hbm.at[pl.ds(b0, BW)], idx_v)
    pltpu.sync_copy(cfl_hbm.at[pl.ds(c0 * N_CLASSES, CW * N_CLASSES)], ctbuf)
    pltpu.sync_copy(xt_hbm.at[pl.ds(c0, CW), pl.ds(b0, BW)], xbuf)

    def c_body(c, a):
        accs = list(a)
        cbase = jnp.zeros((L,), jnp.int32) + c * N_CLASSES
        for b in range(B_VECS):
            lv = idx_v[pl.ds(b * L, L)]
            cv = plsc.load_gather(ctbuf, [cbase + lv])
            xv = xbuf[c, pl.ds(b * L, L)]
            dv = xv - cv
            accs[b % 4] = accs[b % 4] + dv * dv
        return tuple(accs)

    acc = lax.fori_loop(0, CW, c_body, (jnp.zeros((L,), jnp.float32),) * 4)

    accv[...] = (acc[0] + acc[1]) + (acc[2] + acc[3])
    pltpu.sync_copy(accv, out_hbm.at[wid])
    pltpu.sync_copy(xbuf, rel_hbm.at[pl.ds(c0, CW), pl.ds(b0, BW)])


@functools.cache
def _sc_main():
    return pl.kernel(
        _sc_body,
        out_type=(
            jax.ShapeDtypeStruct((NW, L), jnp.float32),
            jax.ShapeDtypeStruct((D, B), jnp.float32),
        ),
        mesh=plsc.VectorSubcoreMesh(core_axis_name="c", subcore_axis_name="s"),
        scratch_types=[
            pltpu.VMEM((BW,), jnp.int32),
            pltpu.VMEM((CW, BW), jnp.float32),
            pltpu.VMEM((CW * N_CLASSES,), jnp.float32),
            pltpu.VMEM((L,), jnp.float32),
        ],
        compiler_params=pltpu.CompilerParams(needs_layout_passes=False),
    )


def _tc_body(c_ref, p_ref, w_ref, l_ref):
    c = c_ref[...]
    sq = jnp.sum(c * c, axis=1)
    g = lax.dot_general(c, c, (((1,), (1,)), ((), ())),
                        preferred_element_type=jnp.float32,
                        precision=lax.Precision.HIGHEST)
    d2 = sq[:, None] + sq[None, :] - 2.0 * g
    dist = jnp.sqrt(jnp.maximum(d2, 0.0))
    gc = jnp.sum(dist, axis=1) * (1.0 / N_CLASSES)
    mx = jnp.max(gc)
    mn = jnp.min(gc)
    w_ref[...] = (gc - mn + EPS) / (mx - mn)
    loss = jnp.sum(p_ref[...]) * (LAMBDA / (B * D))
    l_ref[...] = jnp.reshape(loss, (1, 1))


_tc_pair = pl.pallas_call(
    _tc_body,
    out_shape=(
        jax.ShapeDtypeStruct((N_CLASSES,), jnp.float32),
        jax.ShapeDtypeStruct((1, 1), jnp.float32),
    ),
)


def kernel(relation_logits_raw, rel_labels, centers):
    labels = rel_labels.astype(jnp.int32)
    xt = jnp.swapaxes(relation_logits_raw, 0, 1)
    cflat = jnp.swapaxes(centers, 0, 1).reshape(-1)
    parts, rel_t = _sc_main()(xt, labels, cflat)
    weight, loss = _tc_pair(centers, parts)
    return (weight, loss[0, 0], jnp.swapaxes(rel_t, 0, 1))


# speedup vs baseline: 2.3505x; 1.3578x over previous
import functools

import jax
import jax.numpy as jnp
from jax import lax
from jax.experimental import pallas as pl
from jax.experimental.pallas import tpu as pltpu
from jax.experimental.pallas import tpu_sc as plsc

N_CLASSES = 51
D = 576
B = 4096
LAMBDA = 0.03
EPS = 0.09

NC, NS, L = 2, 16, 16
NW = NC * NS
C_CHUNKS = 8
B_CHUNKS = NW // C_CHUNKS
CW = D // C_CHUNKS
BW = B // B_CHUNKS
N_PHASES = 3
CP = CW // N_PHASES
N_GROUPS = 4
GV = BW // (N_GROUPS * L)


def _sc_body(xt_hbm, lab_hbm, cfl_hbm, out_hbm, rel_hbm,
             idx_v, xbuf, ctbuf, accv,
             sx0, sx1, sx2, sw0, sw1, sw2):
    wid = lax.axis_index("s") * NC + lax.axis_index("c")
    cc = wid % C_CHUNKS
    bc = wid // C_CHUNKS
    c0 = cc * CW
    b0 = bc * BW

    pltpu.sync_copy(lab_hbm.at[pl.ds(b0, BW)], idx_v)
    pltpu.sync_copy(cfl_hbm.at[pl.ds(c0 * N_CLASSES, CW * N_CLASSES)], ctbuf)

    sxs = (sx0, sx1, sx2)
    sws = (sw0, sw1, sw2)
    loads = [
        pltpu.async_copy(
            xt_hbm.at[pl.ds(c0 + p * CP, CP), pl.ds(b0, BW)],
            xbuf.at[pl.ds(p * CP, CP)], sxs[p])
        for p in range(N_PHASES)
    ]

    acc = [jnp.zeros((L,), jnp.float32)] * 4
    wbs = []
    for p in range(N_PHASES):
        loads[p].wait()
        wbs.append(pltpu.async_copy(
            xbuf.at[pl.ds(p * CP, CP)],
            rel_hbm.at[pl.ds(c0 + p * CP, CP), pl.ds(b0, BW)], sws[p]))
        for bg in range(N_GROUPS):
            lvs = [idx_v[pl.ds((bg * GV + j) * L, L)] for j in range(GV)]

            def c_body(c, carry, _bg=bg, _lvs=lvs):
                cvec = carry[0]
                accs = list(carry[1:])
                for j in range(GV):
                    cv = plsc.load_gather(ctbuf, [cvec + _lvs[j]])
                    xv = xbuf[c, pl.ds((_bg * GV + j) * L, L)]
                    dv = xv - cv
                    accs[j % 4] = accs[j % 4] + dv * dv
                return (cvec + N_CLASSES, *accs)

            out = lax.fori_loop(
                p * CP, (p + 1) * CP, c_body,
                (jnp.full((L,), p * CP * N_CLASSES, jnp.int32), *acc))
            acc = list(out[1:])

    accv[...] = (acc[0] + acc[1]) + (acc[2] + acc[3])
    pltpu.sync_copy(accv, out_hbm.at[wid])
    for wb in wbs:
        wb.wait()


@functools.cache
def _sc_main():
    return pl.kernel(
        _sc_body,
        out_type=(
            jax.ShapeDtypeStruct((NW, L), jnp.float32),
            jax.ShapeDtypeStruct((D, B), jnp.float32),
        ),
        mesh=plsc.VectorSubcoreMesh(core_axis_name="c", subcore_axis_name="s"),
        scratch_types=[
            pltpu.VMEM((BW,), jnp.int32),
            pltpu.VMEM((CW, BW), jnp.float32),
            pltpu.VMEM((CW * N_CLASSES,), jnp.float32),
            pltpu.VMEM((L,), jnp.float32),
            pltpu.SemaphoreType.DMA,
            pltpu.SemaphoreType.DMA,
            pltpu.SemaphoreType.DMA,
            pltpu.SemaphoreType.DMA,
            pltpu.SemaphoreType.DMA,
            pltpu.SemaphoreType.DMA,
        ],
        compiler_params=pltpu.CompilerParams(needs_layout_passes=False),
    )


def _tc_pair_body(c_ref, w_ref):
    c = c_ref[...]
    sq = jnp.sum(c * c, axis=1)
    g = lax.dot_general(c, c, (((1,), (1,)), ((), ())),
                        preferred_element_type=jnp.float32,
                        precision=lax.Precision.HIGHEST)
    d2 = sq[:, None] + sq[None, :] - 2.0 * g
    dist = jnp.sqrt(jnp.maximum(d2, 0.0))
    gc = jnp.sum(dist, axis=1) * (1.0 / N_CLASSES)
    mx = jnp.max(gc)
    mn = jnp.min(gc)
    w_ref[...] = (gc - mn + EPS) / (mx - mn)


_tc_pair = pl.pallas_call(
    _tc_pair_body,
    out_shape=jax.ShapeDtypeStruct((N_CLASSES,), jnp.float32),
)


def _tc_loss_body(p_ref, l_ref):
    loss = jnp.sum(p_ref[...]) * (LAMBDA / (B * D))
    l_ref[...] = jnp.reshape(loss, (1, 1))


_tc_loss = pl.pallas_call(
    _tc_loss_body,
    out_shape=jax.ShapeDtypeStruct((1, 1), jnp.float32),
)


def kernel(relation_logits_raw, rel_labels, centers):
    labels = rel_labels.astype(jnp.int32)
    xt = jnp.swapaxes(relation_logits_raw, 0, 1)
    cflat = jnp.swapaxes(centers, 0, 1).reshape(-1)
    parts, rel_t = _sc_main()(xt, labels, cflat)
    weight = _tc_pair(centers)
    loss = _tc_loss(parts)
    return (weight, loss[0, 0], jnp.swapaxes(rel_t, 0, 1))


# speedup vs baseline: 2.3721x; 1.0092x over previous
import functools

import jax
import jax.numpy as jnp
from jax import lax
from jax.experimental import pallas as pl
from jax.experimental.pallas import tpu as pltpu
from jax.experimental.pallas import tpu_sc as plsc

N_CLASSES = 51
D = 576
B = 4096
LAMBDA = 0.03
EPS = 0.09

NC, NS, L = 2, 16, 16
NW = NC * NS
C_CHUNKS = 8
B_CHUNKS = NW // C_CHUNKS
CW = D // C_CHUNKS
BW = B // B_CHUNKS
PHASES = ((0, 8), (8, 32), (40, 32))
N_GROUPS = 4
GV = BW // (N_GROUPS * L)


def _sc_body(xt_hbm, lab_hbm, cfl_hbm, out_hbm, rel_hbm,
             idx_v, xbuf, ctbuf, accv,
             sx0, sx1, sx2, sw0, sw1, sw2):
    wid = lax.axis_index("s") * NC + lax.axis_index("c")
    cc = wid % C_CHUNKS
    bc = wid // C_CHUNKS
    c0 = cc * CW
    b0 = bc * BW

    pltpu.sync_copy(lab_hbm.at[pl.ds(b0, BW)], idx_v)
    pltpu.sync_copy(cfl_hbm.at[pl.ds(c0 * N_CLASSES, CW * N_CLASSES)], ctbuf)

    sxs = (sx0, sx1, sx2)
    sws = (sw0, sw1, sw2)
    loads = [
        pltpu.async_copy(
            xt_hbm.at[pl.ds(c0 + ps, pn), pl.ds(b0, BW)],
            xbuf.at[pl.ds(ps, pn)], sxs[p])
        for p, (ps, pn) in enumerate(PHASES)
    ]

    acc = [jnp.zeros((L,), jnp.float32)] * 4
    wbs = []
    for p, (ps, pn) in enumerate(PHASES):
        loads[p].wait()
        wbs.append(pltpu.async_copy(
            xbuf.at[pl.ds(ps, pn)],
            rel_hbm.at[pl.ds(c0 + ps, pn), pl.ds(b0, BW)], sws[p]))
        for bg in range(N_GROUPS):
            lvs = [idx_v[pl.ds((bg * GV + j) * L, L)] for j in range(GV)]

            def c_body(c, carry, _bg=bg, _lvs=lvs):
                cvec = carry[0]
                accs = list(carry[1:])
                for j in range(GV):
                    cv = plsc.load_gather(ctbuf, [cvec + _lvs[j]])
                    xv = xbuf[c, pl.ds((_bg * GV + j) * L, L)]
                    dv = xv - cv
                    accs[j % 4] = accs[j % 4] + dv * dv
                return (cvec + N_CLASSES, *accs)

            out = lax.fori_loop(
                ps, ps + pn, c_body,
                (jnp.full((L,), ps * N_CLASSES, jnp.int32), *acc))
            acc = list(out[1:])

    accv[...] = (acc[0] + acc[1]) + (acc[2] + acc[3])
    pltpu.sync_copy(accv, out_hbm.at[wid])
    for wb in wbs:
        wb.wait()


@functools.cache
def _sc_main():
    return pl.kernel(
        _sc_body,
        out_type=(
            jax.ShapeDtypeStruct((NW, L), jnp.float32),
            jax.ShapeDtypeStruct((D, B), jnp.float32),
        ),
        mesh=plsc.VectorSubcoreMesh(core_axis_name="c", subcore_axis_name="s"),
        scratch_types=[
            pltpu.VMEM((BW,), jnp.int32),
            pltpu.VMEM((CW, BW), jnp.float32),
            pltpu.VMEM((CW * N_CLASSES,), jnp.float32),
            pltpu.VMEM((L,), jnp.float32),
            pltpu.SemaphoreType.DMA,
            pltpu.SemaphoreType.DMA,
            pltpu.SemaphoreType.DMA,
            pltpu.SemaphoreType.DMA,
            pltpu.SemaphoreType.DMA,
            pltpu.SemaphoreType.DMA,
        ],
        compiler_params=pltpu.CompilerParams(needs_layout_passes=False),
    )


def _tc_pair_body(c_ref, w_ref):
    c = c_ref[...]
    sq = jnp.sum(c * c, axis=1)
    g = lax.dot_general(c, c, (((1,), (1,)), ((), ())),
                        preferred_element_type=jnp.float32,
                        precision=lax.Precision.HIGHEST)
    d2 = sq[:, None] + sq[None, :] - 2.0 * g
    dist = jnp.sqrt(jnp.maximum(d2, 0.0))
    gc = jnp.sum(dist, axis=1) * (1.0 / N_CLASSES)
    mx = jnp.max(gc)
    mn = jnp.min(gc)
    w_ref[...] = (gc - mn + EPS) / (mx - mn)


_tc_pair = pl.pallas_call(
    _tc_pair_body,
    out_shape=jax.ShapeDtypeStruct((N_CLASSES,), jnp.float32),
)


def _tc_loss_body(p_ref, l_ref):
    loss = jnp.sum(p_ref[...]) * (LAMBDA / (B * D))
    l_ref[...] = jnp.reshape(loss, (1, 1))


_tc_loss = pl.pallas_call(
    _tc_loss_body,
    out_shape=jax.ShapeDtypeStruct((1, 1), jnp.float32),
)


def kernel(relation_logits_raw, rel_labels, centers):
    labels = rel_labels.astype(jnp.int32)
    xt = jnp.swapaxes(relation_logits_raw, 0, 1)
    cflat = lax.reshape(centers, (N_CLASSES * D,), dimensions=(1, 0))
    parts, rel_t = _sc_main()(xt, labels, cflat)
    weight = _tc_pair(centers)
    loss = _tc_loss(parts)
    return (weight, loss[0, 0], jnp.swapaxes(rel_t, 0, 1))


# speedup vs baseline: 2.4458x; 1.0311x over previous
import functools

import jax
import jax.numpy as jnp
from jax import lax
from jax.experimental import pallas as pl
from jax.experimental.pallas import tpu as pltpu
from jax.experimental.pallas import tpu_sc as plsc

N_CLASSES = 51
D = 576
B = 4096
LAMBDA = 0.03
EPS = 0.09

NC, NS, L = 2, 16, 16
NW = NC * NS
C_CHUNKS = 8
B_CHUNKS = NW // C_CHUNKS
CW = D // C_CHUNKS
BW = B // B_CHUNKS
PHASES = ((0, 8), (8, 32), (40, 32))
N_GROUPS = 4
GV = BW // (N_GROUPS * L)


def _sc_body(xt_hbm, lab_hbm, cfl_hbm, out_hbm, rel_hbm,
             idx_v, xbuf, ctbuf, accv,
             sx0, sx1, sx2, sw0, sw1, sw2, slab, sct):
    wid = lax.axis_index("s") * NC + lax.axis_index("c")
    cc = wid % C_CHUNKS
    bc = wid // C_CHUNKS
    c0 = cc * CW
    b0 = bc * BW

    cp_lab = pltpu.async_copy(lab_hbm.at[pl.ds(b0, BW)], idx_v, slab)
    cp_ct = pltpu.async_copy(
        cfl_hbm.at[pl.ds(c0 * N_CLASSES, CW * N_CLASSES)], ctbuf, sct)

    sxs = (sx0, sx1, sx2)
    sws = (sw0, sw1, sw2)
    loads = [
        pltpu.async_copy(
            xt_hbm.at[pl.ds(c0 + ps, pn), pl.ds(b0, BW)],
            xbuf.at[pl.ds(ps, pn)], sxs[p])
        for p, (ps, pn) in enumerate(PHASES)
    ]
    cp_lab.wait()
    cp_ct.wait()

    acc = [jnp.zeros((L,), jnp.float32)] * 4
    wbs = []
    for p, (ps, pn) in enumerate(PHASES):
        loads[p].wait()
        wbs.append(pltpu.async_copy(
            xbuf.at[pl.ds(ps, pn)],
            rel_hbm.at[pl.ds(c0 + ps, pn), pl.ds(b0, BW)], sws[p]))
        for bg in range(N_GROUPS):
            lvs = [idx_v[pl.ds((bg * GV + j) * L, L)] for j in range(GV)]

            def c_body(c, carry, _bg=bg, _lvs=lvs):
                cvec = carry[0]
                accs = list(carry[1:])
                for j in range(GV):
                    cv = plsc.load_gather(ctbuf, [cvec + _lvs[j]])
                    xv = xbuf[c, pl.ds((_bg * GV + j) * L, L)]
                    dv = xv - cv
                    accs[j % 4] = accs[j % 4] + dv * dv
                return (cvec + N_CLASSES, *accs)

            out = lax.fori_loop(
                ps, ps + pn, c_body,
                (jnp.full((L,), ps * N_CLASSES, jnp.int32), *acc))
            acc = list(out[1:])

    accv[...] = (acc[0] + acc[1]) + (acc[2] + acc[3])
    pltpu.sync_copy(accv, out_hbm.at[wid])
    for wb in wbs:
        wb.wait()


@functools.cache
def _sc_main():
    return pl.kernel(
        _sc_body,
        out_type=(
            jax.ShapeDtypeStruct((NW, L), jnp.float32),
            jax.ShapeDtypeStruct((D, B), jnp.float32),
        ),
        mesh=plsc.VectorSubcoreMesh(core_axis_name="c", subcore_axis_name="s"),
        scratch_types=[
            pltpu.VMEM((BW,), jnp.int32),
            pltpu.VMEM((CW, BW), jnp.float32),
            pltpu.VMEM((CW * N_CLASSES,), jnp.float32),
            pltpu.VMEM((L,), jnp.float32),
            pltpu.SemaphoreType.DMA,
            pltpu.SemaphoreType.DMA,
            pltpu.SemaphoreType.DMA,
            pltpu.SemaphoreType.DMA,
            pltpu.SemaphoreType.DMA,
            pltpu.SemaphoreType.DMA,
            pltpu.SemaphoreType.DMA,
            pltpu.SemaphoreType.DMA,
        ],
        compiler_params=pltpu.CompilerParams(needs_layout_passes=False),
    )


def _tc_pair_body(c_ref, w_ref):
    c = c_ref[...]
    sq = jnp.sum(c * c, axis=1)
    g = lax.dot_general(c, c, (((1,), (1,)), ((), ())),
                        preferred_element_type=jnp.float32,
                        precision=lax.Precision.HIGHEST)
    d2 = sq[:, None] + sq[None, :] - 2.0 * g
    dist = jnp.sqrt(jnp.maximum(d2, 0.0))
    gc = jnp.sum(dist, axis=1) * (1.0 / N_CLASSES)
    mx = jnp.max(gc)
    mn = jnp.min(gc)
    w_ref[...] = (gc - mn + EPS) / (mx - mn)


_tc_pair = pl.pallas_call(
    _tc_pair_body,
    out_shape=jax.ShapeDtypeStruct((N_CLASSES,), jnp.float32),
)


def _tc_loss_body(p_ref, l_ref):
    loss = jnp.sum(p_ref[...]) * (LAMBDA / (B * D))
    l_ref[...] = jnp.reshape(loss, (1, 1))


_tc_loss = pl.pallas_call(
    _tc_loss_body,
    out_shape=jax.ShapeDtypeStruct((1, 1), jnp.float32),
)


def kernel(relation_logits_raw, rel_labels, centers):
    labels = rel_labels.astype(jnp.int32)
    xt = jnp.swapaxes(relation_logits_raw, 0, 1)
    cflat = lax.reshape(centers, (N_CLASSES * D,), dimensions=(1, 0))
    parts, rel_t = _sc_main()(xt, labels, cflat)
    weight = _tc_pair(centers)
    loss = _tc_loss(parts)
    return (weight, loss[0, 0], jnp.swapaxes(rel_t, 0, 1))
